# baseline (speedup 1.0000x reference)
import jax
import jax.numpy as jnp
from jax.experimental import pallas as pl
from jax.experimental.pallas import tpu as pltpu


def _moe_block(x_ref, W_ref, b_ref, Wg_ref, bg_ref, out_ref, gate_ref):
    e = pl.program_id(1)
    xb = x_ref[...]

    @pl.when(e == 0)
    def _():
        logits = jax.lax.dot_general(
            xb, Wg_ref[...], (((1,), (1,)), ((), ())),
            preferred_element_type=jnp.float32) + bg_ref[...][None, :]
        m = logits.max(axis=1, keepdims=True)
        ex = jnp.exp(logits - m)
        probs = ex / ex.sum(axis=1, keepdims=True)
        col = jax.lax.broadcasted_iota(jnp.int32, probs.shape, 1)
        i1 = jnp.argmax(probs, axis=1)[:, None]
        v1 = jnp.max(probs, axis=1, keepdims=True)
        masked = jnp.where(col == i1, -jnp.inf, probs)
        i2 = jnp.argmax(masked, axis=1)[:, None]
        v2 = jnp.max(masked, axis=1, keepdims=True)
        gate_ref[...] = jnp.where(
            col == i1, v1, jnp.where(col == i2, v2, 0.0))

    h = jax.lax.dot_general(
        xb, W_ref[0], (((1,), (1,)), ((), ())),
        preferred_element_type=jnp.float32) + b_ref[0]
    g = gate_ref[...]
    gcol = jnp.where(
        jax.lax.broadcasted_iota(jnp.int32, g.shape, 1) == e, g, 0.0
    ).sum(axis=1, keepdims=True)
    contrib = gcol * jnp.maximum(h, 0.0)

    @pl.when(e == 0)
    def _():
        out_ref[...] = contrib

    @pl.when(e > 0)
    def _():
        out_ref[...] = out_ref[...] + contrib


@jax.jit
def kernel(x, W, b, Wg, bg):
    N, D = x.shape
    E = W.shape[0]
    T = 2048
    grid = (N // T, E)
    return pl.pallas_call(
        _moe_block,
        grid=grid,
        in_specs=[
            pl.BlockSpec((T, D), lambda t, e: (t, 0)),
            pl.BlockSpec((1, D, D), lambda t, e: (e, 0, 0)),
            pl.BlockSpec((1, 1, D), lambda t, e: (e, 0, 0)),
            pl.BlockSpec((E, D), lambda t, e: (0, 0)),
            pl.BlockSpec((E,), lambda t, e: (0,)),
        ],
        out_specs=pl.BlockSpec((T, D), lambda t, e: (t, 0)),
        out_shape=jax.ShapeDtypeStruct((N, D), x.dtype),
        scratch_shapes=[pltpu.VMEM((T, E), jnp.float32)],
    )(x, W, b.reshape(E, 1, D), Wg, bg)


# speedup vs baseline: 1.3217x; 1.3217x over previous
import jax
import jax.numpy as jnp
from jax.experimental import pallas as pl
from jax.experimental.pallas import tpu as pltpu


def _moe_block(x_ref, W_ref, b_ref, Wg_ref, bg_ref, out_ref, wbf_ref):
    @pl.when(pl.program_id(0) == 0)
    def _():
        wbf_ref[...] = W_ref[...].astype(jnp.bfloat16)

    xb = x_ref[...]
    logits = jax.lax.dot_general(
        xb, Wg_ref[...], (((1,), (1,)), ((), ())),
        preferred_element_type=jnp.float32) + bg_ref[...][None, :]
    m = logits.max(axis=1, keepdims=True)
    ex = jnp.exp(logits - m)
    probs = ex / ex.sum(axis=1, keepdims=True)

    E = probs.shape[1]
    col = jax.lax.broadcasted_iota(jnp.int32, probs.shape, 1)
    i1 = jnp.argmax(probs, axis=1)[:, None]
    v1 = jnp.max(probs, axis=1, keepdims=True)
    masked = jnp.where(col == i1, -jnp.inf, probs)
    i2 = jnp.argmax(masked, axis=1)[:, None]
    v2 = jnp.max(masked, axis=1, keepdims=True)
    gate = jnp.where(col == i1, v1, jnp.where(col == i2, v2, 0.0))

    xb16 = xb.astype(jnp.bfloat16)
    terms = []
    for e in range(E):
        h = jax.lax.dot_general(
            xb16, wbf_ref[e], (((1,), (1,)), ((), ())),
            preferred_element_type=jnp.float32) + b_ref[e][None, :]
        terms.append(gate[:, e][:, None] * jnp.maximum(h, 0.0))
    while len(terms) > 1:
        terms = [terms[i] + terms[i + 1] for i in range(0, len(terms), 2)]
    out_ref[...] = terms[0]


@jax.jit
def kernel(x, W, b, Wg, bg):
    N, D = x.shape
    E = W.shape[0]
    T = 512
    grid = (N // T,)
    return pl.pallas_call(
        _moe_block,
        grid=grid,
        in_specs=[
            pl.BlockSpec((T, D), lambda i: (i, 0)),
            pl.BlockSpec((E, D, D), lambda i: (0, 0, 0)),
            pl.BlockSpec((E, D), lambda i: (0, 0)),
            pl.BlockSpec((E, D), lambda i: (0, 0)),
            pl.BlockSpec((E,), lambda i: (0,)),
        ],
        out_specs=pl.BlockSpec((T, D), lambda i: (i, 0)),
        out_shape=jax.ShapeDtypeStruct((N, D), x.dtype),
        scratch_shapes=[pltpu.VMEM((E, D, D), jnp.bfloat16)],
    )(x, W, b, Wg, bg)


# speedup vs baseline: 1.3240x; 1.0018x over previous
import jax
import jax.numpy as jnp
from jax.experimental import pallas as pl
from jax.experimental.pallas import tpu as pltpu


def _moe_block(x_ref, W_ref, b_ref, Wg_ref, bg_ref, out_ref, wbf_ref):
    @pl.when(pl.program_id(0) == 0)
    def _():
        wbf_ref[...] = W_ref[...].astype(jnp.bfloat16)

    xb = x_ref[...]
    logits = jax.lax.dot_general(
        xb, Wg_ref[...], (((1,), (1,)), ((), ())),
        preferred_element_type=jnp.float32) + bg_ref[...][None, :]
    m = logits.max(axis=1, keepdims=True)
    ex = jnp.exp(logits - m)
    probs = ex / ex.sum(axis=1, keepdims=True)

    E = probs.shape[1]
    col = jax.lax.broadcasted_iota(jnp.int32, probs.shape, 1)
    i1 = jnp.argmax(probs, axis=1)[:, None]
    v1 = jnp.max(probs, axis=1, keepdims=True)
    masked = jnp.where(col == i1, -jnp.inf, probs)
    i2 = jnp.argmax(masked, axis=1)[:, None]
    v2 = jnp.max(masked, axis=1, keepdims=True)
    gate = jnp.where(col == i1, v1, jnp.where(col == i2, v2, 0.0))

    xb16 = xb.astype(jnp.bfloat16)
    acc = jnp.zeros_like(xb)
    for e in range(E):
        h = jax.lax.dot_general(
            xb16, wbf_ref[e], (((1,), (1,)), ((), ())),
            preferred_element_type=jnp.float32) + b_ref[e][None, :]
        acc = acc + gate[:, e][:, None] * jnp.maximum(h, 0.0)
    out_ref[...] = acc


@jax.jit
def kernel(x, W, b, Wg, bg):
    N, D = x.shape
    E = W.shape[0]
    T = 256
    grid = (N // T,)
    return pl.pallas_call(
        _moe_block,
        grid=grid,
        in_specs=[
            pl.BlockSpec((T, D), lambda i: (i, 0)),
            pl.BlockSpec((E, D, D), lambda i: (0, 0, 0)),
            pl.BlockSpec((E, D), lambda i: (0, 0)),
            pl.BlockSpec((E, D), lambda i: (0, 0)),
            pl.BlockSpec((E,), lambda i: (0,)),
        ],
        out_specs=pl.BlockSpec((T, D), lambda i: (i, 0)),
        out_shape=jax.ShapeDtypeStruct((N, D), x.dtype),
        scratch_shapes=[pltpu.VMEM((E, D, D), jnp.bfloat16)],
    )(x, W, b, Wg, bg)


# speedup vs baseline: 1.3903x; 1.0500x over previous
import jax
import jax.numpy as jnp
from jax.experimental import pallas as pl
from jax.experimental.pallas import tpu as pltpu


def _moe_block(x_ref, W_ref, b_ref, Wg_ref, bg_ref, out_ref, wbf_ref):
    @pl.when(pl.program_id(0) == 0)
    def _():
        wbf_ref[...] = W_ref[...].astype(jnp.bfloat16)

    xb = x_ref[...]
    logits = jax.lax.dot_general(
        xb, Wg_ref[...], (((1,), (1,)), ((), ())),
        preferred_element_type=jnp.float32) + bg_ref[...][None, :]
    m = logits.max(axis=1, keepdims=True)
    ex = jnp.exp(logits - m)
    probs = ex / ex.sum(axis=1, keepdims=True)

    E = probs.shape[1]
    col = jax.lax.broadcasted_iota(jnp.int32, probs.shape, 1)
    i1 = jnp.argmax(probs, axis=1)[:, None]
    v1 = jnp.max(probs, axis=1, keepdims=True)
    masked = jnp.where(col == i1, -jnp.inf, probs)
    i2 = jnp.argmax(masked, axis=1)[:, None]
    v2 = jnp.max(masked, axis=1, keepdims=True)
    gate = jnp.where(col == i1, v1, jnp.where(col == i2, v2, 0.0))

    xb16 = xb.astype(jnp.bfloat16)
    acc = jnp.zeros_like(xb)
    for e in range(E):
        h = jax.lax.dot_general(
            xb16, wbf_ref[e], (((1,), (1,)), ((), ())),
            preferred_element_type=jnp.float32) + b_ref[e][None, :]
        acc = acc + gate[:, e][:, None] * jnp.maximum(h, 0.0)
    out_ref[...] = acc


@jax.jit
def kernel(x, W, b, Wg, bg):
    N, D = x.shape
    E = W.shape[0]
    T = 1024
    grid = (N // T,)
    return pl.pallas_call(
        _moe_block,
        grid=grid,
        in_specs=[
            pl.BlockSpec((T, D), lambda i: (i, 0)),
            pl.BlockSpec((E, D, D), lambda i: (0, 0, 0)),
            pl.BlockSpec((E, D), lambda i: (0, 0)),
            pl.BlockSpec((E, D), lambda i: (0, 0)),
            pl.BlockSpec((E,), lambda i: (0,)),
        ],
        out_specs=pl.BlockSpec((T, D), lambda i: (i, 0)),
        out_shape=jax.ShapeDtypeStruct((N, D), x.dtype),
        scratch_shapes=[pltpu.VMEM((E, D, D), jnp.bfloat16)],
    )(x, W, b, Wg, bg)
